# Initial kernel scaffold; baseline (speedup 1.0000x reference)
import jax
import jax.numpy as jnp
from jax.experimental import pallas as pl


def kernel(x, Wq, Wk, Wv, Wo):
    raise NotImplementedError("write your pallas kernel here")


# speedup vs baseline: 10.1898x; 10.1898x over previous
import functools

import jax
import jax.numpy as jnp
from jax.experimental import pallas as pl
from jax.experimental.pallas import tpu as pltpu

D_MODEL = 256
N_HEADS = 8
DH = D_MODEL // N_HEADS
TOP_K_N = 8
NEG_INF = float("-inf")


def _attn_head_kernel(x_ref, wq_ref, wk_ref, wv_ref, wo_ref, out_ref,
                      orig_ref, work_ref):
    h = pl.program_id(0)
    T = x_ref.shape[0]
    x = x_ref[...]
    q = jax.lax.dot_general(x, wq_ref[...], (((1,), (1,)), ((), ())),
                            preferred_element_type=jnp.float32)
    k = jax.lax.dot_general(x, wk_ref[...], (((1,), (1,)), ((), ())),
                            preferred_element_type=jnp.float32)
    v = jax.lax.dot_general(x, wv_ref[...], (((1,), (1,)), ((), ())),
                            preferred_element_type=jnp.float32)
    kt = k.T

    row = jax.lax.broadcasted_iota(jnp.int32, (T, T), 0)
    col = jax.lax.broadcasted_iota(jnp.int32, (T, T), 1)
    causal = col > row

    sc = q[:, 0:1] + kt[0:1, :]
    for d in range(1, DH):
        sc = jnp.maximum(sc, q[:, d:d + 1] + kt[d:d + 1, :])
    sc = jnp.where(causal, NEG_INF, sc)
    orig_ref[...] = sc
    work_ref[...] = sc

    for _ in range(TOP_K_N):
        a = work_ref[...]
        vm = jnp.max(a, axis=1, keepdims=True)
        idx = jnp.min(jnp.where(a == vm, col, T), axis=1,
                      keepdims=True)
        work_ref[...] = jnp.where(col == idx, NEG_INF, a)

    a = work_ref[...]
    canvas = jnp.where(a == NEG_INF, orig_ref[...], NEG_INF)
    vmax = jnp.max(canvas, axis=1, keepdims=True)
    num = jnp.exp(canvas - vmax)
    denom = jnp.sum(num, axis=1, keepdims=True)
    attnw = num * (1.0 / denom)

    ctx = jnp.dot(attnw, v, preferred_element_type=jnp.float32)
    contrib = jnp.dot(ctx, wo_ref[...], preferred_element_type=jnp.float32)

    @pl.when(h == 0)
    def _init():
        out_ref[...] = contrib

    @pl.when(h != 0)
    def _acc():
        out_ref[...] += contrib


@jax.jit
def kernel(x, Wq, Wk, Wv, Wo):
    B, T, D = x.shape
    x2 = x.reshape(B * T, D)
    out = pl.pallas_call(
        _attn_head_kernel,
        grid=(N_HEADS,),
        in_specs=[
            pl.BlockSpec((B * T, D), lambda h: (0, 0)),
            pl.BlockSpec((DH, D), lambda h: (h, 0)),
            pl.BlockSpec((DH, D), lambda h: (h, 0)),
            pl.BlockSpec((DH, D), lambda h: (h, 0)),
            pl.BlockSpec((DH, D), lambda h: (h, 0)),
        ],
        out_specs=pl.BlockSpec((B * T, D), lambda h: (0, 0)),
        out_shape=jax.ShapeDtypeStruct((B * T, D), jnp.float32),
        scratch_shapes=[
            pltpu.VMEM((B * T, B * T), jnp.float32),
            pltpu.VMEM((B * T, B * T), jnp.float32),
        ],
    )(x2, Wq, Wk, Wv, Wo.T)
    return out.reshape(B, T, D)
